# baseline (speedup 1.0000x reference)
import jax
import jax.numpy as jnp
from jax.experimental import pallas as pl
from jax.experimental.pallas import tpu as pltpu


def _pick_block(m: int, max_rows: int) -> int:
    best = 0
    i = 1
    while i * i <= m:
        if m % i == 0:
            for d in (i, m // i):
                if d % 8 == 0 and d <= max_rows and d > best:
                    best = d
        i += 1
    return best


def _pool_body(x_ref, o_ref):
    j = pl.program_id(1)

    @pl.when(j == 0)
    def _():
        o_ref[...] = jnp.zeros_like(o_ref)

    x = x_ref[...]
    r = o_ref.shape[0]
    o_ref[...] += jnp.sum(x.reshape(-1, r, x.shape[1]), axis=0)


def kernel(features, h, w):
    n, c = features.shape
    p = 2 if n % 2 == 0 else 1
    m = n // p
    bn = _pick_block(m, 48_000)
    if bn == 0:
        p, m = 1, n
        bn = _pick_block(m, 65_536) or m
    k = m // bn
    r = 1
    while r < 64 and bn % (r * 2) == 0:
        r *= 2

    partials = pl.pallas_call(
        _pool_body,
        grid=(p, k),
        in_specs=[pl.BlockSpec((bn, c), lambda i, j: (i * k + j, 0))],
        out_specs=pl.BlockSpec((r, c), lambda i, j: (i, 0)),
        out_shape=jax.ShapeDtypeStruct((p * r, c), jnp.float32),
        compiler_params=pltpu.CompilerParams(
            dimension_semantics=("parallel", "arbitrary"),
        ),
    )(features)
    return jnp.sum(partials, axis=0) / (h * w)
